# baseline (speedup 1.0000x reference)
import functools

import jax
import jax.numpy as jnp
from jax import lax
from jax.experimental import pallas as pl
from jax.experimental.pallas import tpu as pltpu
from jax.experimental.pallas import tpu_sc as plsc

NUM_CORES = 2
NUM_SUBCORES = 16
NL = 16
NW = NUM_CORES * NUM_SUBCORES

B = 1024
L = 200
E = 64
V = 1000000

VCH = 384
VMAIN = 999936
NCH1 = VMAIN // VCH
TAILV = 128

BCH = 128
LCH = 8


def _t_common(ib, ob, ncols, iota):
    diags = [(iota + s) & (NL - 1) for s in range(NL)]
    cves = [((d & 1) << 6) + iota for d in diags]
    evs = [iota + e0 for e0 in range(0, E, NL)]

    def ub_body(ub, carry):
        u0 = ub * NL
        u0h = ub * (NL // 2)
        for s in range(NL):
            uvec = u0 + diags[s]
            jvec = u0h + lax.shift_right_logical(diags[s], 1)
            for e0 in range(0, E, NL):
                v = plsc.load_gather(ib, [evs[e0 // NL], uvec])
                plsc.store_scatter(ob, [jvec, cves[s] + e0], v)
        return carry

    lax.fori_loop(0, ncols // NL, ub_body, 0)


def _body1(tt_hbm, tail_hbm, scr_hbm, ib0, ib1, ob0, ob1, sg0, sg1, so0, so1):
    wid = lax.axis_index("s") * NUM_CORES + lax.axis_index("c")
    iota = lax.iota(jnp.int32, NL)
    ibs, obs, sgs, sos = (ib0, ib1), (ob0, ob1), (sg0, sg1), (so0, so1)

    nch = (NCH1 - wid + NW - 1) // NW

    def in_start(c, slot):
        v0 = pl.multiple_of(c * VCH, 128)
        pltpu.async_copy(tt_hbm.at[slice(None), pl.ds(v0, VCH)], ibs[slot], sgs[slot])

    def in_wait(c, slot):
        v0 = pl.multiple_of(c * VCH, 128)
        pltpu.make_async_copy(
            tt_hbm.at[slice(None), pl.ds(v0, VCH)], ibs[slot], sgs[slot]).wait()

    def out_start(c, slot):
        r0 = pl.multiple_of(c * (VCH // 2), 8)
        pltpu.async_copy(obs[slot], scr_hbm.at[pl.ds(r0, VCH // 2)], sos[slot])

    def out_wait(c, slot):
        r0 = pl.multiple_of(c * (VCH // 2), 8)
        pltpu.make_async_copy(
            obs[slot], scr_hbm.at[pl.ds(r0, VCH // 2)], sos[slot]).wait()

    in_start(wid, 0)

    def step(i, slot):
        c = wid + i * NW

        @pl.when(i + 1 < nch)
        def _():
            in_start(c + NW, 1 - slot)

        in_wait(c, slot)

        @pl.when(i >= 2)
        def _():
            out_wait(c - 2 * NW, slot)

        _t_common(ibs[slot], obs[slot], VCH, iota)
        out_start(c, slot)

    def pair_body(p, carry):
        i0 = 2 * p

        @pl.when(i0 < nch)
        def _():
            step(i0, 0)

        @pl.when(i0 + 1 < nch)
        def _():
            step(i0 + 1, 1)

        return carry

    lax.fori_loop(0, (NCH1 // NW + 2) // 2, pair_body, 0)
    out_wait(wid, 0)
    out_wait(wid, 1)

    @pl.when(wid == NW - 1)
    def _():
        pltpu.sync_copy(tail_hbm, ibs[0].at[slice(None), pl.ds(0, TAILV)])
        _t_common(ibs[0], obs[0], TAILV, iota)
        pltpu.sync_copy(obs[0].at[pl.ds(TAILV // 4, TAILV // 4)],
                        scr_hbm.at[pl.ds(VMAIN // 2, TAILV // 4)])


def _body2(xt_hbm, scr_hbm, out_hbm, idxr, idx2, rows0, rows1, out_u,
           sg0, sg1, so):
    wid = lax.axis_index("s") * NUM_CORES + lax.axis_index("c")
    bc = wid % 8
    lr = wid // 8
    b0 = pl.multiple_of(bc * BCH, 128)
    iota = lax.iota(jnp.int32, NL)
    rows = (rows0, rows1)
    sgs = (sg0, sg1)

    def stage_chunk(l0):
        pltpu.sync_copy(xt_hbm.at[pl.ds(pl.multiple_of(l0, 8), LCH),
                                  pl.ds(b0, BCH)], idxr)
        def lrow(j, carry):
            for bq in range(BCH // NL):
                vv = idxr[j, pl.ds(bq * NL, NL)]
                idx2[j, pl.ds(bq * NL, NL)] = lax.shift_right_logical(vv, 1)
            return carry
        lax.fori_loop(0, LCH, lrow, 0)

    def gather_start(j, slot):
        pltpu.async_copy(scr_hbm.at[idx2.at[j]], rows[slot], sgs[slot])

    def gather_wait(j, slot):
        pltpu.make_async_copy(scr_hbm.at[idx2.at[j]], rows[slot], sgs[slot]).wait()

    diags = [(iota + s) & (NL - 1) for s in range(NL)]

    def transpose_l(lp, slot):
        lspl = jnp.full((NL,), 0, jnp.int32) + lp

        def bq_body(bq, carry):
            bvec = iota + bq * NL
            par = lax.shift_left(idxr[lp, pl.ds(bq * NL, NL)] & 1, 6)
            for s in range(NL):
                pd = par + diags[s]
                for e0 in range(0, E, NL):
                    v = plsc.load_gather(rows[slot], [bvec, pd + e0])
                    plsc.store_scatter(out_u, [diags[s] + e0, lspl, bvec], v)
            return carry

        lax.fori_loop(0, BCH // NL, bq_body, 0)

    def chunk_body(k, carry):
        l0 = (lr + 4 * k) * LCH
        stage_chunk(l0)
        gather_start(0, 0)

        def lp_body(lp, carry2):
            l2 = 2 * lp
            gather_wait(l2, 0)
            gather_start(l2 + 1, 1)
            transpose_l(l2, 0)

            @pl.when(lp < LCH // 2 - 1)
            def _():
                gather_start(l2 + 2, 0)

            gather_wait(l2 + 1, 1)
            transpose_l(l2 + 1, 1)
            return carry2

        lax.fori_loop(0, LCH // 2, lp_body, 0)
        pltpu.sync_copy(
            out_u,
            out_hbm.at[slice(None), pl.ds(pl.multiple_of(l0, 8), LCH),
                       pl.ds(b0, BCH)])
        return carry

    nchunk = jnp.where(lr == 0, 7, 6)
    lax.fori_loop(0, nchunk, chunk_body, 0)


@functools.partial(jax.jit, static_argnames=())
def kernel(x, table):
    xt = x.T
    tt = table.T
    tail = lax.slice(tt, (0, V - TAILV), (E, V))
    mesh = plsc.VectorSubcoreMesh(
        core_axis_name="c", subcore_axis_name="s",
        num_cores=NUM_CORES, num_subcores=NUM_SUBCORES)
    cp = pltpu.CompilerParams(use_tc_tiling_on_sc=True, needs_layout_passes=False)

    k1 = pl.kernel(
        _body1,
        out_type=jax.ShapeDtypeStruct((V // 2, 128), jnp.float32),
        mesh=mesh,
        scratch_types=[
            pltpu.VMEM((E, VCH), jnp.float32),
            pltpu.VMEM((E, VCH), jnp.float32),
            pltpu.VMEM((VCH // 2, 128), jnp.float32),
            pltpu.VMEM((VCH // 2, 128), jnp.float32),
            pltpu.SemaphoreType.DMA,
            pltpu.SemaphoreType.DMA,
            pltpu.SemaphoreType.DMA,
            pltpu.SemaphoreType.DMA,
        ],
        compiler_params=cp,
    )
    scr = k1(tt, tail)

    k2 = pl.kernel(
        _body2,
        out_type=jax.ShapeDtypeStruct((E, L, B), jnp.float32),
        mesh=mesh,
        scratch_types=[
            pltpu.VMEM((LCH, BCH), jnp.int32),
            pltpu.VMEM((LCH, BCH), jnp.int32),
            pltpu.VMEM((BCH, 128), jnp.float32),
            pltpu.VMEM((BCH, 128), jnp.float32),
            pltpu.VMEM((E, LCH, BCH), jnp.float32),
            pltpu.SemaphoreType.DMA,
            pltpu.SemaphoreType.DMA,
            pltpu.SemaphoreType.DMA,
        ],
        compiler_params=cp,
    )
    outT = k2(xt, scr)
    return outT.transpose(2, 0, 1)


# speedup vs baseline: 1.0245x; 1.0245x over previous
import functools

import jax
import jax.numpy as jnp
from jax import lax
from jax.experimental import pallas as pl
from jax.experimental.pallas import tpu as pltpu
from jax.experimental.pallas import tpu_sc as plsc

NUM_CORES = 2
NUM_SUBCORES = 16
NL = 16
NW = NUM_CORES * NUM_SUBCORES

B = 1024
L = 200
E = 64
V = 1000000

VCH = 384
VMAIN = 999936
NCH1 = VMAIN // VCH
TAILV = 128

BCH = 128
LCH = 8


def _t_common(ib, ob, ncols, iota):
    def ub_body(ub, carry):
        u0 = ub * NL

        def s_body(s, carry2):
            uvec = u0 + ((iota + s) & (NL - 1))
            jvec = lax.shift_right_logical(uvec, 1)
            pvec = (uvec & 1) << 6
            for e0 in range(0, E, NL):
                evec = iota + e0
                v = plsc.load_gather(ib, [evec, uvec])
                plsc.store_scatter(ob, [jvec, pvec + evec], v)
            return carry2

        lax.fori_loop(0, NL, s_body, 0, unroll=4)
        return carry

    lax.fori_loop(0, ncols // NL, ub_body, 0)


def _body1(tt_hbm, tail_hbm, scr_hbm, ib0, ib1, ob0, ob1, sg0, sg1, so0, so1):
    wid = lax.axis_index("s") * NUM_CORES + lax.axis_index("c")
    iota = lax.iota(jnp.int32, NL)
    ibs, obs, sgs, sos = (ib0, ib1), (ob0, ob1), (sg0, sg1), (so0, so1)

    nch = (NCH1 - wid + NW - 1) // NW

    def in_start(c, slot):
        v0 = pl.multiple_of(c * VCH, 128)
        pltpu.async_copy(tt_hbm.at[slice(None), pl.ds(v0, VCH)], ibs[slot], sgs[slot])

    def in_wait(c, slot):
        v0 = pl.multiple_of(c * VCH, 128)
        pltpu.make_async_copy(
            tt_hbm.at[slice(None), pl.ds(v0, VCH)], ibs[slot], sgs[slot]).wait()

    def out_start(c, slot):
        r0 = pl.multiple_of(c * (VCH // 2), 8)
        pltpu.async_copy(obs[slot], scr_hbm.at[pl.ds(r0, VCH // 2)], sos[slot])

    def out_wait(c, slot):
        r0 = pl.multiple_of(c * (VCH // 2), 8)
        pltpu.make_async_copy(
            obs[slot], scr_hbm.at[pl.ds(r0, VCH // 2)], sos[slot]).wait()

    in_start(wid, 0)

    def step(i, slot):
        c = wid + i * NW

        @pl.when(i + 1 < nch)
        def _():
            in_start(c + NW, 1 - slot)

        in_wait(c, slot)

        @pl.when(i >= 2)
        def _():
            out_wait(c - 2 * NW, slot)

        _t_common(ibs[slot], obs[slot], VCH, iota)
        out_start(c, slot)

    def pair_body(p, carry):
        i0 = 2 * p

        @pl.when(i0 < nch)
        def _():
            step(i0, 0)

        @pl.when(i0 + 1 < nch)
        def _():
            step(i0 + 1, 1)

        return carry

    lax.fori_loop(0, (NCH1 // NW + 2) // 2, pair_body, 0)
    out_wait(wid, 0)
    out_wait(wid, 1)

    @pl.when(wid == NW - 1)
    def _():
        pltpu.sync_copy(tail_hbm, ibs[0].at[slice(None), pl.ds(0, TAILV)])
        _t_common(ibs[0], obs[0], TAILV, iota)
        pltpu.sync_copy(obs[0].at[pl.ds(TAILV // 4, TAILV // 4)],
                        scr_hbm.at[pl.ds(VMAIN // 2, TAILV // 4)])


def _body2(xt_hbm, scr_hbm, out_hbm, idxr, idx2, rows0, rows1, out_u,
           sg0, sg1, so):
    wid = lax.axis_index("s") * NUM_CORES + lax.axis_index("c")
    bc = wid % 8
    lr = wid // 8
    b0 = pl.multiple_of(bc * BCH, 128)
    iota = lax.iota(jnp.int32, NL)
    rows = (rows0, rows1)
    sgs = (sg0, sg1)

    def stage_chunk(l0):
        pltpu.sync_copy(xt_hbm.at[pl.ds(pl.multiple_of(l0, 8), LCH),
                                  pl.ds(b0, BCH)], idxr)
        def lrow(j, carry):
            for bq in range(BCH // NL):
                vv = idxr[j, pl.ds(bq * NL, NL)]
                idx2[j, pl.ds(bq * NL, NL)] = lax.shift_right_logical(vv, 1)
            return carry
        lax.fori_loop(0, LCH, lrow, 0)

    def gather_start(j, slot):
        pltpu.async_copy(scr_hbm.at[idx2.at[j]], rows[slot], sgs[slot])

    def gather_wait(j, slot):
        pltpu.make_async_copy(scr_hbm.at[idx2.at[j]], rows[slot], sgs[slot]).wait()

    diags = [(iota + s) & (NL - 1) for s in range(NL)]

    def transpose_l(lp, slot):
        lspl = jnp.full((NL,), 0, jnp.int32) + lp

        def bq_body(bq, carry):
            bvec = iota + bq * NL
            par = lax.shift_left(idxr[lp, pl.ds(bq * NL, NL)] & 1, 6)
            for s in range(NL):
                pd = par + diags[s]
                for e0 in range(0, E, NL):
                    v = plsc.load_gather(rows[slot], [bvec, pd + e0])
                    plsc.store_scatter(out_u, [diags[s] + e0, lspl, bvec], v)
            return carry

        lax.fori_loop(0, BCH // NL, bq_body, 0)

    def chunk_body(k, carry):
        l0 = (lr + 4 * k) * LCH
        stage_chunk(l0)
        gather_start(0, 0)

        def lp_body(lp, carry2):
            l2 = 2 * lp
            gather_wait(l2, 0)
            gather_start(l2 + 1, 1)
            transpose_l(l2, 0)

            @pl.when(lp < LCH // 2 - 1)
            def _():
                gather_start(l2 + 2, 0)

            gather_wait(l2 + 1, 1)
            transpose_l(l2 + 1, 1)
            return carry2

        lax.fori_loop(0, LCH // 2, lp_body, 0)
        pltpu.sync_copy(
            out_u,
            out_hbm.at[slice(None), pl.ds(pl.multiple_of(l0, 8), LCH),
                       pl.ds(b0, BCH)])
        return carry

    nchunk = jnp.where(lr == 0, 7, 6)
    lax.fori_loop(0, nchunk, chunk_body, 0)


@functools.partial(jax.jit, static_argnames=())
def kernel(x, table):
    xt = x.T
    tt = table.T
    tail = lax.slice(tt, (0, V - TAILV), (E, V))
    mesh = plsc.VectorSubcoreMesh(
        core_axis_name="c", subcore_axis_name="s",
        num_cores=NUM_CORES, num_subcores=NUM_SUBCORES)
    cp = pltpu.CompilerParams(use_tc_tiling_on_sc=True, needs_layout_passes=False)

    k1 = pl.kernel(
        _body1,
        out_type=jax.ShapeDtypeStruct((V // 2, 128), jnp.float32),
        mesh=mesh,
        scratch_types=[
            pltpu.VMEM((E, VCH), jnp.float32),
            pltpu.VMEM((E, VCH), jnp.float32),
            pltpu.VMEM((VCH // 2, 128), jnp.float32),
            pltpu.VMEM((VCH // 2, 128), jnp.float32),
            pltpu.SemaphoreType.DMA,
            pltpu.SemaphoreType.DMA,
            pltpu.SemaphoreType.DMA,
            pltpu.SemaphoreType.DMA,
        ],
        compiler_params=cp,
    )
    scr = k1(tt, tail)

    k2 = pl.kernel(
        _body2,
        out_type=jax.ShapeDtypeStruct((E, L, B), jnp.float32),
        mesh=mesh,
        scratch_types=[
            pltpu.VMEM((LCH, BCH), jnp.int32),
            pltpu.VMEM((LCH, BCH), jnp.int32),
            pltpu.VMEM((BCH, 128), jnp.float32),
            pltpu.VMEM((BCH, 128), jnp.float32),
            pltpu.VMEM((E, LCH, BCH), jnp.float32),
            pltpu.SemaphoreType.DMA,
            pltpu.SemaphoreType.DMA,
            pltpu.SemaphoreType.DMA,
        ],
        compiler_params=cp,
    )
    outT = k2(xt, scr)
    return outT.transpose(2, 0, 1)


# speedup vs baseline: 1.0807x; 1.0548x over previous
import functools

import jax
import jax.numpy as jnp
from jax import lax
from jax.experimental import pallas as pl
from jax.experimental.pallas import tpu as pltpu
from jax.experimental.pallas import tpu_sc as plsc

NUM_CORES = 2
NUM_SUBCORES = 16
NL = 16
NW = NUM_CORES * NUM_SUBCORES

B = 1024
L = 200
E = 64
V = 1000000

VCH = 384
VMAIN = 999936
NCH1 = VMAIN // VCH
TAILV = 128

BCH = 128
LCH = 8


def _t_common(ib, ob, ncols, iota):
    def ub_body(ub, carry):
        u0 = ub * NL

        def s_body(s, carry2):
            uvec = u0 + ((iota + s) & (NL - 1))
            jvec = lax.shift_right_logical(uvec, 1)
            pvec = (uvec & 1) << 6
            for e0 in range(0, E, NL):
                evec = iota + e0
                v = plsc.load_gather(ib, [evec, uvec])
                plsc.store_scatter(ob, [jvec, pvec + evec], v)
            return carry2

        lax.fori_loop(0, NL, s_body, 0, unroll=4)
        return carry

    lax.fori_loop(0, ncols // NL, ub_body, 0)


def _body1(tt_hbm, tail_hbm, scr_hbm, ib0, ib1, ob0, ob1, sg0, sg1, so0, so1):
    wid = lax.axis_index("s") * NUM_CORES + lax.axis_index("c")
    iota = lax.iota(jnp.int32, NL)
    ibs, obs, sgs, sos = (ib0, ib1), (ob0, ob1), (sg0, sg1), (so0, so1)

    nch = (NCH1 - wid + NW - 1) // NW

    def in_start(c, slot):
        v0 = pl.multiple_of(c * VCH, 128)
        pltpu.async_copy(tt_hbm.at[slice(None), pl.ds(v0, VCH)], ibs[slot], sgs[slot])

    def in_wait(c, slot):
        v0 = pl.multiple_of(c * VCH, 128)
        pltpu.make_async_copy(
            tt_hbm.at[slice(None), pl.ds(v0, VCH)], ibs[slot], sgs[slot]).wait()

    def out_start(c, slot):
        r0 = pl.multiple_of(c * (VCH // 2), 8)
        pltpu.async_copy(obs[slot], scr_hbm.at[pl.ds(r0, VCH // 2)], sos[slot])

    def out_wait(c, slot):
        r0 = pl.multiple_of(c * (VCH // 2), 8)
        pltpu.make_async_copy(
            obs[slot], scr_hbm.at[pl.ds(r0, VCH // 2)], sos[slot]).wait()

    in_start(wid, 0)

    def step(i, slot):
        c = wid + i * NW

        @pl.when(i + 1 < nch)
        def _():
            in_start(c + NW, 1 - slot)

        in_wait(c, slot)

        @pl.when(i >= 2)
        def _():
            out_wait(c - 2 * NW, slot)

        _t_common(ibs[slot], obs[slot], VCH, iota)
        out_start(c, slot)

    def pair_body(p, carry):
        i0 = 2 * p

        @pl.when(i0 < nch)
        def _():
            step(i0, 0)

        @pl.when(i0 + 1 < nch)
        def _():
            step(i0 + 1, 1)

        return carry

    lax.fori_loop(0, (NCH1 // NW + 2) // 2, pair_body, 0)
    out_wait(wid, 0)
    out_wait(wid, 1)

    @pl.when(wid == NW - 1)
    def _():
        pltpu.sync_copy(tail_hbm, ibs[0].at[slice(None), pl.ds(0, TAILV)])
        _t_common(ibs[0], obs[0], TAILV, iota)
        pltpu.sync_copy(obs[0].at[pl.ds(TAILV // 4, TAILV // 4)],
                        scr_hbm.at[pl.ds(VMAIN // 2, TAILV // 4)])


def _body2(xt_hbm, scr_hbm, out_hbm, idxr, idx2, rows0, rows1, out_u,
           sg0, sg1, so):
    wid = lax.axis_index("s") * NUM_CORES + lax.axis_index("c")
    bc = wid % 8
    lr = wid // 8
    b0 = pl.multiple_of(bc * BCH, 128)
    iota = lax.iota(jnp.int32, NL)
    rows = (rows0, rows1)
    sgs = (sg0, sg1)

    def stage_chunk(l0):
        pltpu.sync_copy(xt_hbm.at[pl.ds(pl.multiple_of(l0, 8), LCH),
                                  pl.ds(b0, BCH)], idxr)
        def lrow(j, carry):
            for bq in range(BCH // NL):
                vv = idxr[j, pl.ds(bq * NL, NL)]
                idx2[j, pl.ds(bq * NL, NL)] = lax.shift_right_logical(vv, 1)
            return carry
        lax.fori_loop(0, LCH, lrow, 0)

    def gather_start(j, slot):
        pltpu.async_copy(scr_hbm.at[idx2.at[j]], rows[slot], sgs[slot])

    def gather_wait(j, slot):
        pltpu.make_async_copy(scr_hbm.at[idx2.at[j]], rows[slot], sgs[slot]).wait()

    def transpose_l(lp, slot):
        lspl = jnp.full((NL,), 0, jnp.int32) + lp

        def bq_body(bq, carry):
            bvec = iota + bq * NL
            par = lax.shift_left(idxr[lp, pl.ds(bq * NL, NL)] & 1, 6)

            def s_body(s, carry2):
                diag = (iota + s) & (NL - 1)
                for e0 in range(0, E, NL):
                    ev = par + diag + e0
                    v = plsc.load_gather(rows[slot], [bvec, ev])
                    plsc.store_scatter(out_u, [diag + e0, lspl, bvec], v)
                return carry2

            lax.fori_loop(0, NL, s_body, 0, unroll=4)
            return carry

        lax.fori_loop(0, BCH // NL, bq_body, 0)

    def chunk_body(k, carry):
        l0 = (lr + 4 * k) * LCH
        stage_chunk(l0)
        gather_start(0, 0)

        def lp_body(lp, carry2):
            l2 = 2 * lp
            gather_wait(l2, 0)
            gather_start(l2 + 1, 1)
            transpose_l(l2, 0)

            @pl.when(lp < LCH // 2 - 1)
            def _():
                gather_start(l2 + 2, 0)

            gather_wait(l2 + 1, 1)
            transpose_l(l2 + 1, 1)
            return carry2

        lax.fori_loop(0, LCH // 2, lp_body, 0)
        pltpu.sync_copy(
            out_u,
            out_hbm.at[slice(None), pl.ds(pl.multiple_of(l0, 8), LCH),
                       pl.ds(b0, BCH)])
        return carry

    nchunk = jnp.where(lr == 0, 7, 6)
    lax.fori_loop(0, nchunk, chunk_body, 0)


@functools.partial(jax.jit, static_argnames=())
def kernel(x, table):
    xt = x.T
    tt = table.T
    tail = lax.slice(tt, (0, V - TAILV), (E, V))
    mesh = plsc.VectorSubcoreMesh(
        core_axis_name="c", subcore_axis_name="s",
        num_cores=NUM_CORES, num_subcores=NUM_SUBCORES)
    cp = pltpu.CompilerParams(use_tc_tiling_on_sc=True, needs_layout_passes=False)

    k1 = pl.kernel(
        _body1,
        out_type=jax.ShapeDtypeStruct((V // 2, 128), jnp.float32),
        mesh=mesh,
        scratch_types=[
            pltpu.VMEM((E, VCH), jnp.float32),
            pltpu.VMEM((E, VCH), jnp.float32),
            pltpu.VMEM((VCH // 2, 128), jnp.float32),
            pltpu.VMEM((VCH // 2, 128), jnp.float32),
            pltpu.SemaphoreType.DMA,
            pltpu.SemaphoreType.DMA,
            pltpu.SemaphoreType.DMA,
            pltpu.SemaphoreType.DMA,
        ],
        compiler_params=cp,
    )
    scr = k1(tt, tail)

    k2 = pl.kernel(
        _body2,
        out_type=jax.ShapeDtypeStruct((E, L, B), jnp.float32),
        mesh=mesh,
        scratch_types=[
            pltpu.VMEM((LCH, BCH), jnp.int32),
            pltpu.VMEM((LCH, BCH), jnp.int32),
            pltpu.VMEM((BCH, 128), jnp.float32),
            pltpu.VMEM((BCH, 128), jnp.float32),
            pltpu.VMEM((E, LCH, BCH), jnp.float32),
            pltpu.SemaphoreType.DMA,
            pltpu.SemaphoreType.DMA,
            pltpu.SemaphoreType.DMA,
        ],
        compiler_params=cp,
    )
    outT = k2(xt, scr)
    return outT.transpose(2, 0, 1)
